# speedup vs baseline: 1.1433x; 1.1433x over previous
import jax
import jax.numpy as jnp
from jax import lax
from jax.experimental import pallas as pl

SEARCH = 8
TB = 128


def _body(u_t_ref, x_ref, Wp_ref, bp_ref, Ws_ref, Wt_ref, out_ref):
    x = x_ref[...]
    Wp = Wp_ref[...]
    bp = bp_ref[...]
    Ws = Ws_ref[...]
    Wt = Wt_ref[...]

    ic_t = lax.dot_general(Wp, x, (((0,), (2,)), ((), ())),
                           preferred_element_type=jnp.float32)
    ic_t = ic_t + bp[:, :, None]
    tgt_t = jnp.tanh(lax.dot_general(Wt, ic_t, (((0,), (0,)), ((), ())),
                                     preferred_element_type=jnp.float32))

    ssum = u_t_ref[...]
    out_ref[0, :, :] = ssum
    n_iota = lax.broadcasted_iota(jnp.int32, (TB, tgt_t.shape[2]), 1)
    for i in range(SEARCH):
        m = ssum * (1.0 / (i + 1.0))
        src = jnp.tanh(lax.dot_general(Ws, m, (((0,), (0,)), ((), ())),
                                       preferred_element_type=jnp.float32))
        scores = jnp.sum(tgt_t * src[:, :, None], axis=0)
        mx = jnp.max(scores, axis=1, keepdims=True)
        cand = jnp.where(scores == mx, n_iota, jnp.int32(2**30))
        idx = jnp.min(cand, axis=1, keepdims=True)
        onehot = (n_iota == idx).astype(jnp.float32)
        item = jnp.sum(ic_t * onehot[None, :, :], axis=2)
        ssum = ssum + item
        out_ref[i + 1, :, :] = item


def kernel(user_intent, item_corpus, W_proj, b_proj, W_s, W_t):
    B, N, DIN = item_corpus.shape
    H = W_proj.shape[1]
    grid = (B // TB,)
    out = pl.pallas_call(
        _body,
        grid=grid,
        in_specs=[
            pl.BlockSpec((H, TB), lambda g: (0, g)),
            pl.BlockSpec((TB, N, DIN), lambda g: (g, 0, 0)),
            pl.BlockSpec((DIN, H), lambda g: (0, 0)),
            pl.BlockSpec((H, 1), lambda g: (0, 0)),
            pl.BlockSpec((H, H), lambda g: (0, 0)),
            pl.BlockSpec((H, H), lambda g: (0, 0)),
        ],
        out_specs=pl.BlockSpec((SEARCH + 1, H, TB), lambda g: (0, 0, g)),
        out_shape=jax.ShapeDtypeStruct((SEARCH + 1, H, B), jnp.float32),
    )(user_intent.T, item_corpus, W_proj, b_proj.reshape(H, 1), W_s, W_t)
    return jnp.transpose(out, (2, 0, 1))


# speedup vs baseline: 1.3939x; 1.2192x over previous
import jax
import jax.numpy as jnp
from jax import lax
from jax.experimental import pallas as pl

SEARCH = 8
TB = 128


def _body(u_t_ref, x_ref, Wp_ref, bp_ref, Ws_ref, Wt_ref, out_ref):
    x = x_ref[...]
    Wp = Wp_ref[...]
    bp = bp_ref[...]
    Ws = Ws_ref[...]
    Wt = Wt_ref[...]

    ic_t = lax.dot_general(Wp, x, (((0,), (2,)), ((), ())),
                           preferred_element_type=jnp.float32)
    ic_t = ic_t + bp[:, :, None]
    tgt_t = jnp.tanh(lax.dot_general(Wt, ic_t, (((0,), (0,)), ((), ())),
                                     preferred_element_type=jnp.float32))

    ic_a = jnp.swapaxes(ic_t, 1, 2)
    tgt_a = jnp.swapaxes(tgt_t, 1, 2)
    N = ic_a.shape[1]

    ssum = u_t_ref[...]
    out_ref[0, :, :] = ssum
    n_iota = lax.broadcasted_iota(jnp.int32, (N, TB), 0)
    for i in range(SEARCH):
        m = ssum * (1.0 / (i + 1.0))
        src = jnp.tanh(lax.dot_general(Ws, m, (((0,), (0,)), ((), ())),
                                       preferred_element_type=jnp.float32))
        scores = jnp.sum(tgt_a * src[:, None, :], axis=0)
        mx = jnp.max(scores, axis=0, keepdims=True)
        cand = jnp.where(scores == mx, n_iota, jnp.int32(2**30))
        idx = jnp.min(cand, axis=0, keepdims=True)
        onehot = (n_iota == idx).astype(jnp.float32)
        item = jnp.sum(ic_a * onehot[None, :, :], axis=1)
        ssum = ssum + item
        out_ref[i + 1, :, :] = item


def kernel(user_intent, item_corpus, W_proj, b_proj, W_s, W_t):
    B, N, DIN = item_corpus.shape
    H = W_proj.shape[1]
    grid = (B // TB,)
    out = pl.pallas_call(
        _body,
        grid=grid,
        in_specs=[
            pl.BlockSpec((H, TB), lambda g: (0, g)),
            pl.BlockSpec((TB, N, DIN), lambda g: (g, 0, 0)),
            pl.BlockSpec((DIN, H), lambda g: (0, 0)),
            pl.BlockSpec((H, 1), lambda g: (0, 0)),
            pl.BlockSpec((H, H), lambda g: (0, 0)),
            pl.BlockSpec((H, H), lambda g: (0, 0)),
        ],
        out_specs=pl.BlockSpec((SEARCH + 1, H, TB), lambda g: (0, 0, g)),
        out_shape=jax.ShapeDtypeStruct((SEARCH + 1, H, B), jnp.float32),
    )(user_intent.T, item_corpus, W_proj, b_proj.reshape(H, 1), W_s, W_t)
    return jnp.transpose(out, (2, 0, 1))
